# Initial kernel scaffold; baseline (speedup 1.0000x reference)
import jax
import jax.numpy as jnp
from jax.experimental import pallas as pl


def kernel(x, edge_index, W1, b1, W2, b2):
    raise NotImplementedError("write your pallas kernel here")


# speedup vs baseline: 10.4947x; 10.4947x over previous
import functools

import jax
import jax.numpy as jnp
from jax import lax
from jax.experimental import pallas as pl
from jax.experimental.pallas import tpu as pltpu, tpu_sc as plsc

N = 10000
E = 320000
NC, NS, L = 2, 16, 16
NW = NC * NS
CHUNK = 128
CPT = 80
EPAD = NW * CPT * CHUNK - E
SINK = N
ROWS = 10240
RPT = ROWS // NS
F32 = jnp.float32


def _sc_deg(dst3):
    mesh = plsc.VectorSubcoreMesh(core_axis_name="c", subcore_axis_name="s")

    @functools.partial(
        pl.kernel,
        out_type=jax.ShapeDtypeStruct((NC, ROWS), F32),
        mesh=mesh,
        compiler_params=pltpu.CompilerParams(use_tc_tiling_on_sc=False),
        scratch_types=[
            pltpu.VMEM((CPT, CHUNK), jnp.int32),
            pltpu.VMEM((CHUNK,), F32),
            pltpu.VMEM((RPT,), F32),
            pltpu.VMEM_SHARED((ROWS,), F32),
        ],
    )
    def k(dst_hbm, out_hbm, dstv, ones_v, zbuf, deg_sh):
        c = lax.axis_index("c")
        s = lax.axis_index("s")
        wid = c * NS + s
        one16 = jnp.full((L,), 1.0, F32)
        z16 = jnp.zeros((L,), F32)
        for i in range(CHUNK // L):
            ones_v[pl.ds(i * L, L)] = one16
        for i in range(RPT // L):
            zbuf[pl.ds(i * L, L)] = z16
        pltpu.sync_copy(zbuf, deg_sh.at[pl.ds(s * RPT, RPT)])
        pltpu.sync_copy(dst_hbm.at[wid], dstv)
        plsc.subcore_barrier()

        def body(j, carry):
            pltpu.sync_copy(ones_v, deg_sh.at[dstv.at[j]], add=True)
            return carry

        lax.fori_loop(0, CPT, body, 0)
        plsc.subcore_barrier()
        pltpu.sync_copy(deg_sh.at[pl.ds(s * RPT, RPT)],
                        out_hbm.at[c, pl.ds(s * RPT, RPT)])

    return k(dst3)


def _sc_agg(hp, src3, dst3, D):
    mesh = plsc.VectorSubcoreMesh(core_axis_name="c", subcore_axis_name="s")

    @functools.partial(
        pl.kernel,
        out_type=jax.ShapeDtypeStruct((NC, ROWS, D), F32),
        mesh=mesh,
        compiler_params=pltpu.CompilerParams(use_tc_tiling_on_sc=False),
        scratch_types=[
            pltpu.VMEM((CPT, CHUNK), jnp.int32),
            pltpu.VMEM((CPT, CHUNK), jnp.int32),
            pltpu.VMEM((CHUNK, D), F32),
            pltpu.VMEM((L, D), F32),
            pltpu.VMEM_SHARED((ROWS, D), F32),
            pltpu.SemaphoreType.DMA,
        ],
    )
    def k(hp_hbm, src_hbm, dst_hbm, out_hbm, srcv, dstv, rows, zbuf, acc_sh, sem):
        c = lax.axis_index("c")
        s = lax.axis_index("s")
        wid = c * NS + s
        z16 = jnp.zeros((L,), F32)
        for r in range(L):
            for i in range(D // L):
                zbuf[r, pl.ds(i * L, L)] = z16

        def zbody(m, carry):
            pltpu.sync_copy(zbuf, acc_sh.at[pl.ds(s * RPT + m * L, L)])
            return carry

        lax.fori_loop(0, RPT // L, zbody, 0)
        pltpu.sync_copy(src_hbm.at[wid], srcv)
        pltpu.sync_copy(dst_hbm.at[wid], dstv)
        plsc.subcore_barrier()

        def body(j, carry):
            pltpu.async_copy(hp_hbm.at[srcv.at[j]], rows, sem).wait()
            pltpu.sync_copy(rows, acc_sh.at[dstv.at[j]], add=True)
            return carry

        lax.fori_loop(0, CPT, body, 0)
        plsc.subcore_barrier()
        pltpu.sync_copy(acc_sh.at[pl.ds(s * RPT, RPT)],
                        out_hbm.at[c, pl.ds(s * RPT, RPT)])

    return k(hp, src3, dst3)


def _tc_mm1(x, W1, degp3):
    NB, BLK = 10, N // 10
    F, H = W1.shape

    def body(deg_ref, w_ref, x_ref, h_ref, dinv_ref):
        deg = deg_ref[0] + deg_ref[1] + 1.0
        dinv = lax.rsqrt(deg)
        h = jnp.dot(x_ref[...], w_ref[...], preferred_element_type=F32,
                    precision=lax.Precision.HIGHEST)
        h_ref[...] = h * dinv
        dinv_ref[...] = dinv

    return pl.pallas_call(
        body,
        grid=(NB,),
        in_specs=[
            pl.BlockSpec((2, BLK, 1), lambda i: (0, i, 0)),
            pl.BlockSpec((F, H), lambda i: (0, 0)),
            pl.BlockSpec((BLK, F), lambda i: (i, 0)),
        ],
        out_specs=[
            pl.BlockSpec((BLK, H), lambda i: (i, 0)),
            pl.BlockSpec((BLK, 1), lambda i: (i, 0)),
        ],
        out_shape=[
            jax.ShapeDtypeStruct((N, H), F32),
            jax.ShapeDtypeStruct((N, 1), F32),
        ],
    )(degp3, W1, x)


def _tc_mm2(dinv, P, h1p, W2, b1r):
    NB, BLK = 10, N // 10
    H, O = W2.shape

    def body(dinv_ref, p_ref, h1p_ref, w_ref, b_ref, out_ref):
        dinv = dinv_ref[...]
        agg = p_ref[0] + p_ref[1] + h1p_ref[...]
        y = jnp.maximum(dinv * agg + b_ref[...], 0.0)
        out_ref[...] = jnp.dot(y, w_ref[...], preferred_element_type=F32,
                               precision=lax.Precision.HIGHEST) * dinv

    return pl.pallas_call(
        body,
        grid=(NB,),
        in_specs=[
            pl.BlockSpec((BLK, 1), lambda i: (i, 0)),
            pl.BlockSpec((2, BLK, H), lambda i: (0, i, 0)),
            pl.BlockSpec((BLK, H), lambda i: (i, 0)),
            pl.BlockSpec((H, O), lambda i: (0, 0)),
            pl.BlockSpec((1, H), lambda i: (0, 0)),
        ],
        out_specs=pl.BlockSpec((BLK, O), lambda i: (i, 0)),
        out_shape=jax.ShapeDtypeStruct((N, O), F32),
    )(dinv, P, h1p, W2, b1r)


def _tc_final(dinv, Q, h2p, b2r):
    O = h2p.shape[1]

    def body(dinv_ref, q_ref, h2p_ref, b_ref, out_ref):
        t = dinv_ref[...] * (q_ref[0] + q_ref[1] + h2p_ref[...]) + b_ref[...]
        mean = jnp.sum(t, axis=0, keepdims=True) * (1.0 / N)
        d = t - mean
        var = jnp.sum(d * d, axis=0, keepdims=True) * (1.0 / (N - 1))
        out_ref[...] = d * (10.0 / jnp.sqrt(var))

    return pl.pallas_call(
        body,
        grid=(1,),
        in_specs=[
            pl.BlockSpec((N, 1), lambda i: (0, 0)),
            pl.BlockSpec((2, N, O), lambda i: (0, 0, 0)),
            pl.BlockSpec((N, O), lambda i: (0, 0)),
            pl.BlockSpec((1, O), lambda i: (0, 0)),
        ],
        out_specs=pl.BlockSpec((N, O), lambda i: (0, 0)),
        out_shape=jax.ShapeDtypeStruct((N, O), F32),
    )(dinv, Q, h2p, b2r)


def kernel(x, edge_index, W1, b1, W2, b2):
    src = edge_index[0].astype(jnp.int32)
    dst = edge_index[1].astype(jnp.int32)
    src3 = jnp.concatenate([src, jnp.zeros((EPAD,), jnp.int32)]).reshape(NW, CPT, CHUNK)
    dst3 = jnp.concatenate([dst, jnp.full((EPAD,), SINK, jnp.int32)]).reshape(NW, CPT, CHUNK)

    degp = _sc_deg(dst3)
    h1p, dinv = _tc_mm1(x, W1, degp.reshape(NC, ROWS, 1))
    P = _sc_agg(h1p, src3, dst3, 128)
    h2p = _tc_mm2(dinv, P, h1p, W2, b1.reshape(1, -1))
    Q = _sc_agg(h2p, src3, dst3, 64)
    return _tc_final(dinv, Q, h2p, b2.reshape(1, -1))


# speedup vs baseline: 14.0959x; 1.3431x over previous
import functools

import jax
import jax.numpy as jnp
from jax import lax
from jax.experimental import pallas as pl
from jax.experimental.pallas import tpu as pltpu, tpu_sc as plsc

N = 10000
E = 320000
NC, NS, L = 2, 16, 16
NW = NC * NS
CHUNK = 128
CPT = 80
EPAD = NW * CPT * CHUNK - E
SINK = N
ROWS = 10240
RPT = ROWS // NS
F32 = jnp.float32


def _sc_deg(dst3):
    mesh = plsc.VectorSubcoreMesh(core_axis_name="c", subcore_axis_name="s")

    @functools.partial(
        pl.kernel,
        out_type=jax.ShapeDtypeStruct((NC, ROWS), F32),
        mesh=mesh,
        compiler_params=pltpu.CompilerParams(use_tc_tiling_on_sc=False),
        scratch_types=[
            pltpu.VMEM((CPT, CHUNK), jnp.int32),
            pltpu.VMEM((CHUNK,), F32),
            pltpu.VMEM((RPT,), F32),
            pltpu.VMEM_SHARED((ROWS,), F32),
        ],
    )
    def k(dst_hbm, out_hbm, dstv, ones_v, zbuf, deg_sh):
        c = lax.axis_index("c")
        s = lax.axis_index("s")
        wid = c * NS + s
        one16 = jnp.full((L,), 1.0, F32)
        z16 = jnp.zeros((L,), F32)
        for i in range(CHUNK // L):
            ones_v[pl.ds(i * L, L)] = one16
        for i in range(RPT // L):
            zbuf[pl.ds(i * L, L)] = z16
        pltpu.sync_copy(zbuf, deg_sh.at[pl.ds(s * RPT, RPT)])
        pltpu.sync_copy(dst_hbm.at[wid], dstv)
        plsc.subcore_barrier()

        def body(j, carry):
            pltpu.sync_copy(ones_v, deg_sh.at[dstv.at[j]], add=True)
            return carry

        lax.fori_loop(0, CPT, body, 0)
        plsc.subcore_barrier()
        pltpu.sync_copy(deg_sh.at[pl.ds(s * RPT, RPT)],
                        out_hbm.at[c, pl.ds(s * RPT, RPT)])

    return k(dst3)


def _sc_agg(hp, src_r, dst_r, split):
    R, A = 5, 3
    D = 64
    CP = src_r.shape[1]
    assert CP % R == 0
    mesh = plsc.VectorSubcoreMesh(core_axis_name="c", subcore_axis_name="s")

    @functools.partial(
        pl.kernel,
        out_type=jax.ShapeDtypeStruct((NC, ROWS, D), F32),
        mesh=mesh,
        compiler_params=pltpu.CompilerParams(use_tc_tiling_on_sc=False),
        scratch_types=[
            pltpu.VMEM((CP, CHUNK), jnp.int32),
            pltpu.VMEM((CP, CHUNK), jnp.int32),
            pltpu.VMEM((R, CHUNK, D), F32),
            pltpu.VMEM((L, D), F32),
            pltpu.VMEM_SHARED((ROWS, D), F32),
            [pltpu.SemaphoreType.DMA] * R,
            [pltpu.SemaphoreType.DMA] * R,
        ],
    )
    def k(hp_hbm, src_hbm, dst_hbm, out_hbm, srcv, dstv, rows, zbuf, acc_sh,
          gsem, ssem):
        c = lax.axis_index("c")
        s = lax.axis_index("s")
        blk = s if split else c * NS + s
        gref = hp_hbm.at[c] if split else hp_hbm
        z16 = jnp.zeros((L,), F32)
        for r in range(L):
            for i in range(D // L):
                zbuf[r, pl.ds(i * L, L)] = z16

        def zbody(m, carry):
            pltpu.sync_copy(zbuf, acc_sh.at[pl.ds(s * RPT + m * L, L)])
            return carry

        lax.fori_loop(0, RPT // L, zbody, 0)
        pltpu.sync_copy(src_hbm.at[blk], srcv)
        pltpu.sync_copy(dst_hbm.at[blk], dstv)
        plsc.subcore_barrier()

        for b in range(A):
            pltpu.async_copy(gref.at[srcv.at[b]], rows.at[b], gsem[b])

        def body(i, carry):
            for r in range(R):
                j = i * R + r
                rn = (r + A) % R
                pltpu.make_async_copy(gref.at[pl.ds(0, CHUNK)], rows.at[r],
                                      gsem[r]).wait()
                pltpu.async_copy(rows.at[r], acc_sh.at[dstv.at[j]], ssem[r],
                                 add=True)
                @pl.when(j >= R - A)
                def _():
                    pltpu.make_async_copy(rows.at[rn],
                                          acc_sh.at[pl.ds(0, CHUNK)],
                                          ssem[rn]).wait()

                @pl.when(j < CP - A)
                def _():
                    pltpu.async_copy(gref.at[srcv.at[j + A]], rows.at[rn],
                                     gsem[rn])
            return carry

        lax.fori_loop(0, CP // R, body, 0)
        for b in range(R - A):
            r = (CP - (R - A) + b) % R
            pltpu.make_async_copy(rows.at[r], acc_sh.at[pl.ds(0, CHUNK)],
                                  ssem[r]).wait()
        plsc.subcore_barrier()
        pltpu.sync_copy(acc_sh.at[pl.ds(s * RPT, RPT)],
                        out_hbm.at[c, pl.ds(s * RPT, RPT)])

    return k(hp, src_r, dst_r)


def _tc_mm1(x, W1, degp3):
    NB, BLK = 10, N // 10
    F, H = W1.shape

    def body(deg_ref, w_ref, x_ref, h_ref, dinv_ref):
        deg = deg_ref[0] + deg_ref[1] + 1.0
        dinv = lax.rsqrt(deg)
        h = jnp.dot(x_ref[...], w_ref[...], preferred_element_type=F32,
                    precision=lax.Precision.HIGHEST) * dinv
        h_ref[0] = h[:, :H // 2]
        h_ref[1] = h[:, H // 2:]
        dinv_ref[...] = dinv

    return pl.pallas_call(
        body,
        grid=(NB,),
        in_specs=[
            pl.BlockSpec((2, BLK, 1), lambda i: (0, i, 0)),
            pl.BlockSpec((F, H), lambda i: (0, 0)),
            pl.BlockSpec((BLK, F), lambda i: (i, 0)),
        ],
        out_specs=[
            pl.BlockSpec((2, BLK, H // 2), lambda i: (0, i, 0)),
            pl.BlockSpec((BLK, 1), lambda i: (i, 0)),
        ],
        out_shape=[
            jax.ShapeDtypeStruct((2, N, H // 2), F32),
            jax.ShapeDtypeStruct((N, 1), F32),
        ],
    )(degp3, W1, x)


def _tc_mm2(dinv, P, h1p, W2, b1r):
    NB, BLK = 10, N // 10
    H, O = W2.shape

    def body(dinv_ref, p_ref, h1p_ref, w_ref, b_ref, out_ref):
        dinv = dinv_ref[...]
        agg = jnp.concatenate(
            [p_ref[0] + h1p_ref[0], p_ref[1] + h1p_ref[1]], axis=-1)
        y = jnp.maximum(dinv * agg + b_ref[...], 0.0)
        out_ref[...] = jnp.dot(y, w_ref[...], preferred_element_type=F32,
                               precision=lax.Precision.HIGHEST) * dinv

    return pl.pallas_call(
        body,
        grid=(NB,),
        in_specs=[
            pl.BlockSpec((BLK, 1), lambda i: (i, 0)),
            pl.BlockSpec((2, BLK, H // 2), lambda i: (0, i, 0)),
            pl.BlockSpec((2, BLK, H // 2), lambda i: (0, i, 0)),
            pl.BlockSpec((H, O), lambda i: (0, 0)),
            pl.BlockSpec((1, H), lambda i: (0, 0)),
        ],
        out_specs=pl.BlockSpec((BLK, O), lambda i: (i, 0)),
        out_shape=jax.ShapeDtypeStruct((N, O), F32),
    )(dinv, P, h1p, W2, b1r)


def _tc_final(dinv, Q, h2p, b2r):
    O = h2p.shape[1]

    def body(dinv_ref, q_ref, h2p_ref, b_ref, out_ref):
        t = dinv_ref[...] * (q_ref[0] + q_ref[1] + h2p_ref[...]) + b_ref[...]
        mean = jnp.sum(t, axis=0, keepdims=True) * (1.0 / N)
        d = t - mean
        var = jnp.sum(d * d, axis=0, keepdims=True) * (1.0 / (N - 1))
        out_ref[...] = d * (10.0 / jnp.sqrt(var))

    return pl.pallas_call(
        body,
        grid=(1,),
        in_specs=[
            pl.BlockSpec((N, 1), lambda i: (0, 0)),
            pl.BlockSpec((2, N, O), lambda i: (0, 0, 0)),
            pl.BlockSpec((N, O), lambda i: (0, 0)),
            pl.BlockSpec((1, O), lambda i: (0, 0)),
        ],
        out_specs=pl.BlockSpec((N, O), lambda i: (0, 0)),
        out_shape=jax.ShapeDtypeStruct((N, O), F32),
    )(dinv, Q, h2p, b2r)


def kernel(x, edge_index, W1, b1, W2, b2):
    src = jnp.concatenate([edge_index[0].astype(jnp.int32),
                           jnp.zeros((EPAD,), jnp.int32)])
    dst = jnp.concatenate([edge_index[1].astype(jnp.int32),
                           jnp.full((EPAD,), SINK, jnp.int32)])
    src3 = src.reshape(NW, CPT, CHUNK)
    dst3 = dst.reshape(NW, CPT, CHUNK)
    src2 = src.reshape(NS, 2 * CPT, CHUNK)
    dst2 = dst.reshape(NS, 2 * CPT, CHUNK)

    degp = _sc_deg(dst3)
    h1p, dinv = _tc_mm1(x, W1, degp.reshape(NC, ROWS, 1))
    P = _sc_agg(h1p, src2, dst2, split=True)
    h2p = _tc_mm2(dinv, P, h1p, W2, b1.reshape(1, -1))
    Q = _sc_agg(h2p, src3, dst3, split=False)
    return _tc_final(dinv, Q, h2p, b2.reshape(1, -1))


# speedup vs baseline: 28.9421x; 2.0532x over previous
import functools

import jax
import jax.numpy as jnp
from jax import lax
from jax.experimental import pallas as pl
from jax.experimental.pallas import tpu as pltpu, tpu_sc as plsc

N = 10000
E = 320000
NC, NS, L = 2, 16, 16
NW = NC * NS
CHUNK = 128
CPT = 80
EPAD = NW * CPT * CHUNK - E
SINK = N
ROWS = 10240
RPT = ROWS // NS
F32 = jnp.float32


def _sc_deg(dst3):
    mesh = plsc.VectorSubcoreMesh(core_axis_name="c", subcore_axis_name="s")

    @functools.partial(
        pl.kernel,
        out_type=jax.ShapeDtypeStruct((NC, ROWS), F32),
        mesh=mesh,
        compiler_params=pltpu.CompilerParams(use_tc_tiling_on_sc=False),
        scratch_types=[
            pltpu.VMEM((CPT, CHUNK), jnp.int32),
            pltpu.VMEM((CHUNK,), F32),
            pltpu.VMEM((RPT,), F32),
            pltpu.VMEM_SHARED((ROWS,), F32),
        ],
    )
    def k(dst_hbm, out_hbm, dstv, ones_v, zbuf, deg_sh):
        c = lax.axis_index("c")
        s = lax.axis_index("s")
        wid = c * NS + s
        one16 = jnp.full((L,), 1.0, F32)
        z16 = jnp.zeros((L,), F32)
        for i in range(CHUNK // L):
            ones_v[pl.ds(i * L, L)] = one16
        for i in range(RPT // L):
            zbuf[pl.ds(i * L, L)] = z16
        pltpu.sync_copy(zbuf, deg_sh.at[pl.ds(s * RPT, RPT)])
        pltpu.sync_copy(dst_hbm.at[wid], dstv)
        plsc.subcore_barrier()

        def body(j, carry):
            pltpu.sync_copy(ones_v, deg_sh.at[dstv.at[j]], add=True)
            return carry

        lax.fori_loop(0, CPT, body, 0)
        plsc.subcore_barrier()
        pltpu.sync_copy(deg_sh.at[pl.ds(s * RPT, RPT)],
                        out_hbm.at[c, pl.ds(s * RPT, RPT)])

    return k(dst3)


def _sc_agg(hp, src_r, dst_r, split):
    R, A = 4, 2
    D = 64
    NPT = N // NS
    CP = src_r.shape[1]
    CPQ = CP // 4
    assert CPQ % R == 0
    mesh = plsc.VectorSubcoreMesh(core_axis_name="c", subcore_axis_name="s")

    @functools.partial(
        pl.kernel,
        out_type=jax.ShapeDtypeStruct((NC, ROWS, D), F32),
        mesh=mesh,
        compiler_params=pltpu.CompilerParams(use_tc_tiling_on_sc=False),
        scratch_types=[
            pltpu.VMEM((CPQ, CHUNK), jnp.int32),
            pltpu.VMEM((CPQ, CHUNK), jnp.int32),
            pltpu.VMEM((R, CHUNK, D), F32),
            pltpu.VMEM((L, D), F32),
            pltpu.VMEM_SHARED((N, D), F32),
            pltpu.VMEM_SHARED((ROWS, D), F32),
            [pltpu.SemaphoreType.DMA] * R,
            [pltpu.SemaphoreType.DMA] * R,
        ],
    )
    def k(hp_hbm, src_hbm, dst_hbm, out_hbm, srcv, dstv, rows, zbuf, hp_sh,
          acc_sh, gsem, ssem):
        c = lax.axis_index("c")
        s = lax.axis_index("s")
        blk = s if split else c * NS + s
        ghbm = hp_hbm.at[c] if split else hp_hbm
        z16 = jnp.zeros((L,), F32)
        for r in range(L):
            for i in range(D // L):
                zbuf[r, pl.ds(i * L, L)] = z16

        def zbody(m, carry):
            pltpu.sync_copy(zbuf, acc_sh.at[pl.ds(s * RPT + m * L, L)])
            return carry

        lax.fori_loop(0, RPT // L, zbody, 0)
        pltpu.sync_copy(ghbm.at[pl.ds(s * NPT, NPT)],
                        hp_sh.at[pl.ds(s * NPT, NPT)])
        plsc.subcore_barrier()

        for q in range(4):
            pltpu.sync_copy(src_hbm.at[blk, pl.ds(q * CPQ, CPQ)], srcv)
            pltpu.sync_copy(dst_hbm.at[blk, pl.ds(q * CPQ, CPQ)], dstv)
            for b in range(A):
                pltpu.async_copy(hp_sh.at[srcv.at[b]], rows.at[b], gsem[b])

            def body(i, carry):
                for r in range(R):
                    j = i * R + r
                    rn = (r + A) % R
                    pltpu.make_async_copy(hp_sh.at[pl.ds(0, CHUNK)],
                                          rows.at[r], gsem[r]).wait()
                    pltpu.async_copy(rows.at[r], acc_sh.at[dstv.at[j]],
                                     ssem[r], add=True)
                    @pl.when(j >= R - A)
                    def _():
                        pltpu.make_async_copy(rows.at[rn],
                                              acc_sh.at[pl.ds(0, CHUNK)],
                                              ssem[rn]).wait()

                    @pl.when(j < CPQ - A)
                    def _():
                        pltpu.async_copy(hp_sh.at[srcv.at[j + A]],
                                         rows.at[rn], gsem[rn])
                return carry

            lax.fori_loop(0, CPQ // R, body, 0)
            for b in range(R - A):
                r = (CPQ - (R - A) + b) % R
                pltpu.make_async_copy(rows.at[r], acc_sh.at[pl.ds(0, CHUNK)],
                                      ssem[r]).wait()
        plsc.subcore_barrier()
        pltpu.sync_copy(acc_sh.at[pl.ds(s * RPT, RPT)],
                        out_hbm.at[c, pl.ds(s * RPT, RPT)])

    return k(hp, src_r, dst_r)


def _tc_mm1(x, W1, degp3):
    NB, BLK = 10, N // 10
    F, H = W1.shape

    def body(deg_ref, w_ref, x_ref, h_ref, dinv_ref):
        deg = deg_ref[0] + deg_ref[1] + 1.0
        dinv = lax.rsqrt(deg)
        h = jnp.dot(x_ref[...], w_ref[...], preferred_element_type=F32,
                    precision=lax.Precision.HIGHEST) * dinv
        h_ref[0] = h[:, :H // 2]
        h_ref[1] = h[:, H // 2:]
        dinv_ref[...] = dinv

    return pl.pallas_call(
        body,
        grid=(NB,),
        in_specs=[
            pl.BlockSpec((2, BLK, 1), lambda i: (0, i, 0)),
            pl.BlockSpec((F, H), lambda i: (0, 0)),
            pl.BlockSpec((BLK, F), lambda i: (i, 0)),
        ],
        out_specs=[
            pl.BlockSpec((2, BLK, H // 2), lambda i: (0, i, 0)),
            pl.BlockSpec((BLK, 1), lambda i: (i, 0)),
        ],
        out_shape=[
            jax.ShapeDtypeStruct((2, N, H // 2), F32),
            jax.ShapeDtypeStruct((N, 1), F32),
        ],
    )(degp3, W1, x)


def _tc_mm2(dinv, P, h1p, W2, b1r):
    NB, BLK = 10, N // 10
    H, O = W2.shape

    def body(dinv_ref, p_ref, h1p_ref, w_ref, b_ref, out_ref):
        dinv = dinv_ref[...]
        agg = jnp.concatenate(
            [p_ref[0] + h1p_ref[0], p_ref[1] + h1p_ref[1]], axis=-1)
        y = jnp.maximum(dinv * agg + b_ref[...], 0.0)
        out_ref[...] = jnp.dot(y, w_ref[...], preferred_element_type=F32,
                               precision=lax.Precision.HIGHEST) * dinv

    return pl.pallas_call(
        body,
        grid=(NB,),
        in_specs=[
            pl.BlockSpec((BLK, 1), lambda i: (i, 0)),
            pl.BlockSpec((2, BLK, H // 2), lambda i: (0, i, 0)),
            pl.BlockSpec((2, BLK, H // 2), lambda i: (0, i, 0)),
            pl.BlockSpec((H, O), lambda i: (0, 0)),
            pl.BlockSpec((1, H), lambda i: (0, 0)),
        ],
        out_specs=pl.BlockSpec((BLK, O), lambda i: (i, 0)),
        out_shape=jax.ShapeDtypeStruct((N, O), F32),
    )(dinv, P, h1p, W2, b1r)


def _tc_final(dinv, Q, h2p, b2r):
    O = h2p.shape[1]

    def body(dinv_ref, q_ref, h2p_ref, b_ref, out_ref):
        t = dinv_ref[...] * (q_ref[0] + q_ref[1] + h2p_ref[...]) + b_ref[...]
        mean = jnp.sum(t, axis=0, keepdims=True) * (1.0 / N)
        d = t - mean
        var = jnp.sum(d * d, axis=0, keepdims=True) * (1.0 / (N - 1))
        out_ref[...] = d * (10.0 / jnp.sqrt(var))

    return pl.pallas_call(
        body,
        grid=(1,),
        in_specs=[
            pl.BlockSpec((N, 1), lambda i: (0, 0)),
            pl.BlockSpec((2, N, O), lambda i: (0, 0, 0)),
            pl.BlockSpec((N, O), lambda i: (0, 0)),
            pl.BlockSpec((1, O), lambda i: (0, 0)),
        ],
        out_specs=pl.BlockSpec((N, O), lambda i: (0, 0)),
        out_shape=jax.ShapeDtypeStruct((N, O), F32),
    )(dinv, Q, h2p, b2r)


def kernel(x, edge_index, W1, b1, W2, b2):
    src = jnp.concatenate([edge_index[0].astype(jnp.int32),
                           jnp.zeros((EPAD,), jnp.int32)])
    dst = jnp.concatenate([edge_index[1].astype(jnp.int32),
                           jnp.full((EPAD,), SINK, jnp.int32)])
    src3 = src.reshape(NW, CPT, CHUNK)
    dst3 = dst.reshape(NW, CPT, CHUNK)
    src2 = src.reshape(NS, 2 * CPT, CHUNK)
    dst2 = dst.reshape(NS, 2 * CPT, CHUNK)

    degp = _sc_deg(dst3)
    h1p, dinv = _tc_mm1(x, W1, degp.reshape(NC, ROWS, 1))
    P = _sc_agg(h1p, src2, dst2, split=True)
    h2p = _tc_mm2(dinv, P, h1p, W2, b1.reshape(1, -1))
    Q = _sc_agg(h2p, src3, dst3, split=False)
    return _tc_final(dinv, Q, h2p, b2.reshape(1, -1))
